# baseline (speedup 1.0000x reference)
import functools

import jax
import jax.numpy as jnp
from jax.experimental import pallas as pl
from jax.experimental.pallas import tpu as pltpu
from jax.experimental.pallas import tpu_sc as plsc

BB, DD, HH, KK = 8192, 1024, 256, 8
NCH = 2
BH = BB // NCH
TILE = 256
NTH = BH // TILE
TILEC = 1024
NTCH = BH // TILEC
SPLIT = 4
SUBD = DD // SPLIT
NROWSH = BH * SPLIT
_SC_WIN = 128
_NWH = BH // _SC_WIN


def _classify_body(x_ref, wc_ref, bc_ref, a_ref, rank_ref, cnt_ref):
    x_t = x_ref[...]
    logits = jnp.dot(x_t, wc_ref[...], preferred_element_type=jnp.float32)
    logits = logits + bc_ref[...]
    m = jnp.max(logits, axis=1, keepdims=True)
    lane = jax.lax.broadcasted_iota(jnp.int32, (TILEC, KK), 1)
    amax = jnp.min(jnp.where(logits == m, lane, KK), axis=1, keepdims=True)
    onehot = (lane == amax).astype(jnp.float32)
    ri = jax.lax.broadcasted_iota(jnp.int32, (TILEC, TILEC), 0)
    ci = jax.lax.broadcasted_iota(jnp.int32, (TILEC, TILEC), 1)
    lbd = ((ci < ri) & (ci // TILE == ri // TILE)).astype(jnp.float32)
    ranks = jax.lax.dot(lbd, onehot)
    csum = jnp.sum(onehot.reshape(TILEC // TILE, TILE, KK), axis=1)
    cr = jax.lax.broadcasted_iota(jnp.int32, (TILEC // TILE,) * 2, 0)
    cc = jax.lax.broadcasted_iota(jnp.int32, (TILEC // TILE,) * 2, 1)
    lt4 = (cc < cr).astype(jnp.float32)
    cbase = jax.lax.dot(lt4, csum)
    base_tok = jnp.broadcast_to(cbase[:, None, :],
                                (TILEC // TILE, TILE, KK)).reshape(TILEC, KK)
    rank_tok = jnp.sum((ranks + base_tok) * onehot, axis=1, keepdims=True)
    a_ref[...] = amax
    rank_ref[...] = rank_tok
    cnt_ref[...] = jnp.sum(onehot, axis=0, keepdims=True)[None]


def _classify(x, wc, bc, h):
    return pl.pallas_call(
        _classify_body,
        grid=(NTCH,),
        in_specs=[
            pl.BlockSpec((TILEC, DD), lambda t: (h * NTCH + t, 0)),
            pl.BlockSpec((DD, KK), lambda t: (0, 0)),
            pl.BlockSpec((1, KK), lambda t: (0, 0)),
        ],
        out_specs=[
            pl.BlockSpec((TILEC, 1), lambda t: (t, 0)),
            pl.BlockSpec((TILEC, 1), lambda t: (t, 0)),
            pl.BlockSpec((1, 1, KK), lambda t: (t, 0, 0)),
        ],
        out_shape=[
            jax.ShapeDtypeStruct((BH, 1), jnp.int32),
            jax.ShapeDtypeStruct((BH, 1), jnp.float32),
            jax.ShapeDtypeStruct((NTCH, 1, KK), jnp.float32),
        ],
    )(x, wc, bc)


def _destiny_body(a_ref, rank_ref, cnt_ref, dest4_ref, offs_ref):
    cnts = cnt_ref[:, 0, :]
    ri = jax.lax.broadcasted_iota(jnp.int32, (NTCH, NTCH), 0)
    ci = jax.lax.broadcasted_iota(jnp.int32, (NTCH, NTCH), 1)
    ltri = (ci < ri).astype(jnp.float32)
    carry = jax.lax.dot(ltri, cnts,
                        precision=jax.lax.Precision.HIGHEST)
    tot = jnp.sum(cnts, axis=0, keepdims=True)
    er = jax.lax.broadcasted_iota(jnp.int32, (KK, KK), 0)
    ec = jax.lax.broadcasted_iota(jnp.int32, (KK, KK), 1)
    xtri = (er < ec).astype(jnp.float32)
    offs = jax.lax.dot(tot, xtri,
                       precision=jax.lax.Precision.HIGHEST)
    v = offs[None] + carry[:, None, :]
    vtok = jnp.broadcast_to(v, (NTCH, TILEC, KK)).reshape(BH, KK)
    a_t = a_ref[...]
    lane = jax.lax.broadcasted_iota(jnp.int32, (BH, KK), 1)
    onehot = (lane == a_t).astype(jnp.float32)
    base = jnp.sum(onehot * vtok, axis=1, keepdims=True)
    dest = (base + rank_ref[...]).astype(jnp.int32)
    sub = jax.lax.broadcasted_iota(jnp.int32, (BH, SPLIT), 1)
    dest4_ref[...] = dest + sub * BH
    offs_ref[...] = offs.astype(jnp.int32)


def _destiny(a, rank, cnt):
    return pl.pallas_call(
        _destiny_body,
        in_specs=[
            pl.BlockSpec((BH, 1), lambda: (0, 0)),
            pl.BlockSpec((BH, 1), lambda: (0, 0)),
            pl.BlockSpec((NTCH, 1, KK), lambda: (0, 0, 0)),
        ],
        out_specs=[
            pl.BlockSpec((BH, SPLIT), lambda: (0, 0)),
            pl.BlockSpec((1, KK), lambda: (0, 0)),
        ],
        out_shape=[
            jax.ShapeDtypeStruct((BH, SPLIT), jnp.int32),
            jax.ShapeDtypeStruct((1, KK), jnp.int32),
        ],
    )(a, rank, cnt)


def _sc_scatter(x, dest_row, h):
    mesh = plsc.VectorSubcoreMesh(core_axis_name="core",
                                  subcore_axis_name="subcore")

    @functools.partial(
        pl.kernel,
        out_type=jax.ShapeDtypeStruct((NROWSH, SUBD), jnp.float32),
        mesh=mesh)
    def run(x_hbm, i_hbm, o_hbm):
        def body(x_vmem, i_vmem):
            pltpu.sync_copy(x_vmem, o_hbm.at[i_vmem.at[0]])

        pltpu.emit_pipeline(
            body,
            grid=(SPLIT, _NWH),
            in_specs=[
                pl.BlockSpec((_SC_WIN, SUBD), lambda c, w: (h * _NWH + w, c)),
                pl.BlockSpec((1, _SC_WIN), lambda c, w: (0, c * _NWH + w)),
            ],
            out_specs=[],
            core_axis_name=("core", "subcore"),
            dimension_semantics=(pltpu.PARALLEL, pltpu.PARALLEL),
        )(x_hbm, i_hbm)

    return run(x, dest_row)


def _ae_body(offs_ref, x0_ref, x1_ref, x2_ref, x3_ref,
             w1_ref, b1_ref, w2_ref, b2_ref,
             r0_ref, r1_ref, r2_ref, r3_ref):
    t = pl.program_id(0)
    row0 = t * TILE
    rows = jax.lax.broadcasted_iota(jnp.int32, (TILE, 1), 0)
    x_refs = (x0_ref, x1_ref, x2_ref, x3_ref)
    r_refs = (r0_ref, r1_ref, r2_ref, r3_ref)
    for e in range(KK):
        s = jnp.clip(offs_ref[e] - row0, 0, TILE)
        en = jnp.clip(offs_ref[e + 1] - row0, 0, TILE)

        @pl.when(en > s)
        def _():
            h = jnp.dot(x_refs[0][...], w1_ref[e, 0 * SUBD:1 * SUBD, :],
                        preferred_element_type=jnp.float32)
            for cq in range(1, SPLIT):
                h = h + jnp.dot(x_refs[cq][...],
                                w1_ref[e, cq * SUBD:(cq + 1) * SUBD, :],
                                preferred_element_type=jnp.float32)
            h = jax.nn.relu(h + b1_ref[e][None, :])
            mask = (rows >= s) & (rows < en)
            full = (s == 0) & (en == TILE)
            for cp in range(SPLIT):
                r_cp = jnp.dot(h, w2_ref[e, :, cp * SUBD:(cp + 1) * SUBD],
                               preferred_element_type=jnp.float32)
                r_cp = r_cp + b2_ref[e, cp * SUBD:(cp + 1) * SUBD][None, :]

                @pl.when(full)
                def _fullw(cp=cp, r_cp=r_cp):
                    r_refs[cp][...] = r_cp

                @pl.when(jnp.logical_not(full))
                def _partw(cp=cp, r_cp=r_cp):
                    r_refs[cp][...] = jnp.where(mask, r_cp, r_refs[cp][...])


def _grouped_ae(offs9, xs4, w1, b1, w2, b2):
    grid_spec = pltpu.PrefetchScalarGridSpec(
        num_scalar_prefetch=1,
        grid=(NTH,),
        in_specs=[
            pl.BlockSpec((TILE, SUBD), lambda t, offs: (0 * NTH + t, 0)),
            pl.BlockSpec((TILE, SUBD), lambda t, offs: (1 * NTH + t, 0)),
            pl.BlockSpec((TILE, SUBD), lambda t, offs: (2 * NTH + t, 0)),
            pl.BlockSpec((TILE, SUBD), lambda t, offs: (3 * NTH + t, 0)),
            pl.BlockSpec((KK, DD, HH), lambda t, offs: (0, 0, 0)),
            pl.BlockSpec((KK, HH), lambda t, offs: (0, 0)),
            pl.BlockSpec((KK, HH, DD), lambda t, offs: (0, 0, 0)),
            pl.BlockSpec((KK, DD), lambda t, offs: (0, 0)),
        ],
        out_specs=[
            pl.BlockSpec((TILE, SUBD), lambda t, offs: (t, 0)),
            pl.BlockSpec((TILE, SUBD), lambda t, offs: (t, 0)),
            pl.BlockSpec((TILE, SUBD), lambda t, offs: (t, 0)),
            pl.BlockSpec((TILE, SUBD), lambda t, offs: (t, 0)),
        ],
    )
    return pl.pallas_call(
        _ae_body,
        grid_spec=grid_spec,
        out_shape=[jax.ShapeDtypeStruct((BH, SUBD), jnp.float32)] * SPLIT,
    )(offs9, xs4, xs4, xs4, xs4, w1, b1, w2, b2)


def _sc_combine(rs_a, rs_b, dest_a, dest_b):
    mesh = plsc.VectorSubcoreMesh(core_axis_name="core",
                                  subcore_axis_name="subcore")

    @functools.partial(
        pl.kernel,
        out_type=jax.ShapeDtypeStruct((BB, DD), jnp.float32),
        mesh=mesh)
    def run(a0, a1, a2, a3, b0, b1_, b2_, b3, ia_hbm, ib_hbm, o_hbm):
        for half, (refs, i_hbm) in enumerate(
                (((a0, a1, a2, a3), ia_hbm), ((b0, b1_, b2_, b3), ib_hbm))):
            for cplane, r_hbm in enumerate(refs):
                def body(i_vmem, o_vmem, r_hbm=r_hbm):
                    pltpu.sync_copy(r_hbm.at[i_vmem.at[0]], o_vmem)

                pltpu.emit_pipeline(
                    body,
                    grid=(_NWH,),
                    in_specs=[pl.BlockSpec((1, _SC_WIN), lambda w: (0, w))],
                    out_specs=[pl.BlockSpec(
                        (_SC_WIN, SUBD),
                        lambda w, half=half, cplane=cplane:
                            (half * _NWH + w, cplane))],
                    core_axis_name=("core", "subcore"),
                    dimension_semantics=(pltpu.PARALLEL,),
                )(i_hbm, o_hbm)

    return run(*rs_a, *rs_b, dest_a, dest_b)


def kernel(x, W1, b1, W2, b2, Wc, bc):
    bc2 = bc.reshape(1, KK)
    end = jnp.array([BH], jnp.int32)
    outs = []
    for h in range(NCH):
        a, rank, cnt = _classify(x, Wc, bc2, h)
        dest4, offs = _destiny(a, rank, cnt)
        offs9 = jnp.concatenate([offs.reshape(KK), end])
        dest_row = dest4.T.reshape(1, NROWSH)
        dest_plain = dest4[:, 0].reshape(1, BH)
        xs4 = _sc_scatter(x, dest_row, h)
        rs = _grouped_ae(offs9, xs4, W1, b1, W2, b2)
        outs.append((rs, dest_plain, a))
    x_out = _sc_combine(outs[0][0], outs[1][0], outs[0][1], outs[1][1])
    assignment = jnp.concatenate(
        [o[2].reshape(BH) for o in outs])
    return (x_out, assignment)


# speedup vs baseline: 1.0818x; 1.0818x over previous
import functools

import jax
import jax.numpy as jnp
from jax.experimental import pallas as pl
from jax.experimental.pallas import tpu as pltpu
from jax.experimental.pallas import tpu_sc as plsc

BB, DD, HH, KK = 8192, 1024, 256, 8
TILE = 256
NT = BB // TILE
TILEC = 1024
NTC = BB // TILEC
SPLIT = 4
SUBD = DD // SPLIT
NROWS = BB * SPLIT


def _classify_body(x_ref, wc_ref, bc_ref, lbd_ref, a_ref, rank_ref, cnt_ref):
    x_t = x_ref[...]
    logits = jnp.dot(x_t, wc_ref[...], preferred_element_type=jnp.float32)
    logits = logits + bc_ref[...]
    m = jnp.max(logits, axis=1, keepdims=True)
    lane = jax.lax.broadcasted_iota(jnp.int32, (TILEC, KK), 1)
    amax = jnp.min(jnp.where(logits == m, lane, KK), axis=1, keepdims=True)
    onehot = (lane == amax).astype(jnp.float32)
    ranks = jax.lax.dot(lbd_ref[...], onehot)
    csum = jnp.sum(onehot.reshape(TILEC // TILE, TILE, KK), axis=1)
    cr = jax.lax.broadcasted_iota(jnp.int32, (TILEC // TILE,) * 2, 0)
    cc = jax.lax.broadcasted_iota(jnp.int32, (TILEC // TILE,) * 2, 1)
    lt4 = (cc < cr).astype(jnp.float32)
    cbase = jax.lax.dot(lt4, csum)
    base_tok = jnp.broadcast_to(cbase[:, None, :],
                                (TILEC // TILE, TILE, KK)).reshape(TILEC, KK)
    rank_tok = jnp.sum((ranks + base_tok) * onehot, axis=1, keepdims=True)
    a_ref[...] = amax
    rank_ref[...] = rank_tok
    cnt_ref[...] = jnp.sum(onehot, axis=0, keepdims=True)[None]


def _classify(x, wc, bc, lbd):
    return pl.pallas_call(
        _classify_body,
        grid=(NTC,),
        in_specs=[
            pl.BlockSpec((TILEC, DD), lambda t: (t, 0)),
            pl.BlockSpec((DD, KK), lambda t: (0, 0)),
            pl.BlockSpec((1, KK), lambda t: (0, 0)),
            pl.BlockSpec((TILEC, TILEC), lambda t: (0, 0)),
        ],
        out_specs=[
            pl.BlockSpec((TILEC, 1), lambda t: (t, 0)),
            pl.BlockSpec((TILEC, 1), lambda t: (t, 0)),
            pl.BlockSpec((1, 1, KK), lambda t: (t, 0, 0)),
        ],
        out_shape=[
            jax.ShapeDtypeStruct((BB, 1), jnp.int32),
            jax.ShapeDtypeStruct((BB, 1), jnp.float32),
            jax.ShapeDtypeStruct((NTC, 1, KK), jnp.float32),
        ],
    )(x, wc, bc, lbd)


def _destiny_body(a_ref, rank_ref, cnt_ref, dest4_ref, offs_ref):
    cnts = cnt_ref[:, 0, :]
    ri = jax.lax.broadcasted_iota(jnp.int32, (NTC, NTC), 0)
    ci = jax.lax.broadcasted_iota(jnp.int32, (NTC, NTC), 1)
    ltri = (ci < ri).astype(jnp.float32)
    carry = jax.lax.dot(ltri, cnts,
                        precision=jax.lax.Precision.HIGHEST)
    tot = jnp.sum(cnts, axis=0, keepdims=True)
    er = jax.lax.broadcasted_iota(jnp.int32, (KK, KK), 0)
    ec = jax.lax.broadcasted_iota(jnp.int32, (KK, KK), 1)
    xtri = (er < ec).astype(jnp.float32)
    offs = jax.lax.dot(tot, xtri,
                       precision=jax.lax.Precision.HIGHEST)
    v = offs[None] + carry[:, None, :]
    vtok = jnp.broadcast_to(v, (NTC, TILEC, KK)).reshape(BB, KK)
    a_t = a_ref[...]
    lane = jax.lax.broadcasted_iota(jnp.int32, (BB, KK), 1)
    onehot = (lane == a_t).astype(jnp.float32)
    base = jnp.sum(onehot * vtok, axis=1, keepdims=True)
    dest = (base + rank_ref[...]).astype(jnp.int32)
    sub = jax.lax.broadcasted_iota(jnp.int32, (BB, SPLIT), 1)
    dest4_ref[...] = dest + sub * BB
    lane16 = jax.lax.broadcasted_iota(jnp.int32, (1, 16), 1)
    offs16 = jnp.where(lane16 < KK,
                       jnp.pad(offs, ((0, 0), (0, 8))), BB).astype(jnp.int32)
    offs_ref[...] = offs16


def _destiny(a, rank, cnt):
    return pl.pallas_call(
        _destiny_body,
        in_specs=[
            pl.BlockSpec((BB, 1), lambda: (0, 0)),
            pl.BlockSpec((BB, 1), lambda: (0, 0)),
            pl.BlockSpec((NTC, 1, KK), lambda: (0, 0, 0)),
        ],
        out_specs=[
            pl.BlockSpec((BB, SPLIT), lambda: (0, 0)),
            pl.BlockSpec((1, 16), lambda: (0, 0)),
        ],
        out_shape=[
            jax.ShapeDtypeStruct((BB, SPLIT), jnp.int32),
            jax.ShapeDtypeStruct((1, 16), jnp.int32),
        ],
    )(a, rank, cnt)


_SC_WIN = 128
_NW = BB // _SC_WIN
SUBI = DD // 2 // SPLIT


def _sc_scatter(x, dest_row):
    mesh = plsc.VectorSubcoreMesh(core_axis_name="core",
                                  subcore_axis_name="subcore")

    @functools.partial(
        pl.kernel,
        out_type=jax.ShapeDtypeStruct((NROWS, SUBD), jnp.float32),
        mesh=mesh)
    def run(x_hbm, i_hbm, o_hbm):
        def body(x_vmem, i_vmem):
            pltpu.sync_copy(x_vmem, o_hbm.at[i_vmem.at[0]])

        pltpu.emit_pipeline(
            body,
            grid=(SPLIT, _NW),
            in_specs=[
                pl.BlockSpec((_SC_WIN, SUBD), lambda c, w: (w, c)),
                pl.BlockSpec((1, _SC_WIN), lambda c, w: (0, c * _NW + w)),
            ],
            out_specs=[],
            core_axis_name=("core", "subcore"),
            dimension_semantics=(pltpu.PARALLEL, pltpu.PARALLEL),
        )(x_hbm, i_hbm)

    return run(x, dest_row)


def _sc_gather(r0, r1, r2, r3, dest_plain):
    mesh = plsc.VectorSubcoreMesh(core_axis_name="core",
                                  subcore_axis_name="subcore")

    @functools.partial(
        pl.kernel,
        out_type=jax.ShapeDtypeStruct((BB, DD), jnp.float32),
        mesh=mesh)
    def run(r0_hbm, r1_hbm, r2_hbm, r3_hbm, i_hbm, o_hbm):
        for cplane, r_hbm in enumerate((r0_hbm, r1_hbm, r2_hbm, r3_hbm)):
            def body(i_vmem, o_vmem, r_hbm=r_hbm):
                pltpu.sync_copy(r_hbm.at[i_vmem.at[0]], o_vmem)

            pltpu.emit_pipeline(
                body,
                grid=(_NW,),
                in_specs=[pl.BlockSpec((1, _SC_WIN), lambda w: (0, w))],
                out_specs=[pl.BlockSpec((_SC_WIN, SUBD),
                                        lambda w, cplane=cplane: (w, cplane))],
                core_axis_name=("core", "subcore"),
                dimension_semantics=(pltpu.PARALLEL,),
            )(i_hbm, o_hbm)

    return run(r0, r1, r2, r3, dest_plain)


def _ae_body(offs_ref, x0_ref, x1_ref, x2_ref, x3_ref,
             w1_ref, b1_ref, w2_ref, b2_ref,
             r0_ref, r1_ref, r2_ref, r3_ref):
    t = pl.program_id(0)
    row0 = t * TILE
    rows = jax.lax.broadcasted_iota(jnp.int32, (TILE, 1), 0)
    x_refs = (x0_ref, x1_ref, x2_ref, x3_ref)
    r_refs = (r0_ref, r1_ref, r2_ref, r3_ref)
    for e in range(KK):
        s = jnp.clip(offs_ref[e] - row0, 0, TILE)
        en = jnp.clip(offs_ref[e + 1] - row0, 0, TILE)

        @pl.when(en > s)
        def _():
            h = jnp.dot(x_refs[0][...], w1_ref[e, 0 * SUBD:1 * SUBD, :],
                        preferred_element_type=jnp.float32)
            for cq in range(1, SPLIT):
                h = h + jnp.dot(x_refs[cq][...],
                                w1_ref[e, cq * SUBD:(cq + 1) * SUBD, :],
                                preferred_element_type=jnp.float32)
            h = jax.nn.relu(h + b1_ref[e][None, :])
            mask = (rows >= s) & (rows < en)
            full = (s == 0) & (en == TILE)
            for cp in range(SPLIT):
                r_cp = jnp.dot(h, w2_ref[e, :, cp * SUBD:(cp + 1) * SUBD],
                               preferred_element_type=jnp.float32)
                r_cp = r_cp + b2_ref[e, cp * SUBD:(cp + 1) * SUBD][None, :]

                @pl.when(full)
                def _fullw(cp=cp, r_cp=r_cp):
                    r_refs[cp][...] = r_cp

                @pl.when(jnp.logical_not(full))
                def _partw(cp=cp, r_cp=r_cp):
                    r_refs[cp][...] = jnp.where(mask, r_cp, r_refs[cp][...])


def _grouped_ae(offs9, xs4, w1, b1, w2, b2):
    grid_spec = pltpu.PrefetchScalarGridSpec(
        num_scalar_prefetch=1,
        grid=(NT,),
        in_specs=[
            pl.BlockSpec((TILE, SUBD), lambda t, offs: (0 * NT + t, 0)),
            pl.BlockSpec((TILE, SUBD), lambda t, offs: (1 * NT + t, 0)),
            pl.BlockSpec((TILE, SUBD), lambda t, offs: (2 * NT + t, 0)),
            pl.BlockSpec((TILE, SUBD), lambda t, offs: (3 * NT + t, 0)),
            pl.BlockSpec((KK, DD, HH), lambda t, offs: (0, 0, 0)),
            pl.BlockSpec((KK, HH), lambda t, offs: (0, 0)),
            pl.BlockSpec((KK, HH, DD), lambda t, offs: (0, 0, 0)),
            pl.BlockSpec((KK, DD), lambda t, offs: (0, 0)),
        ],
        out_specs=[
            pl.BlockSpec((TILE, SUBD), lambda t, offs: (t, 0)),
            pl.BlockSpec((TILE, SUBD), lambda t, offs: (t, 0)),
            pl.BlockSpec((TILE, SUBD), lambda t, offs: (t, 0)),
            pl.BlockSpec((TILE, SUBD), lambda t, offs: (t, 0)),
        ],
    )
    return pl.pallas_call(
        _ae_body,
        grid_spec=grid_spec,
        out_shape=[jax.ShapeDtypeStruct((BB, SUBD), jnp.float32)] * SPLIT,
    )(offs9, xs4, xs4, xs4, xs4, w1, b1, w2, b2)


def kernel(x, W1, b1, W2, b2, Wc, bc):
    ri = jax.lax.broadcasted_iota(jnp.int32, (TILEC, TILEC), 0)
    ci = jax.lax.broadcasted_iota(jnp.int32, (TILEC, TILEC), 1)
    lbd = ((ci < ri) & (ci // TILE == ri // TILE)).astype(jnp.float32)
    a, rank, cnt = _classify(x, Wc, bc.reshape(1, KK), lbd)
    dest4, offs16 = _destiny(a, rank, cnt)
    dest_row = dest4.T.reshape(1, NROWS)
    dest_plain = dest4[:, 0].reshape(1, BB)
    xs4 = _sc_scatter(x, dest_row)
    r0, r1, r2, r3 = _grouped_ae(offs16.reshape(16), xs4, W1, b1, W2, b2)
    x_out = _sc_gather(r0, r1, r2, r3, dest_plain)
    return (x_out, a.reshape(BB))


# speedup vs baseline: 1.1489x; 1.0621x over previous
import functools

import jax
import jax.numpy as jnp
from jax.experimental import pallas as pl
from jax.experimental.pallas import tpu as pltpu
from jax.experimental.pallas import tpu_sc as plsc

BB, DD, HH, KK = 8192, 1024, 256, 8
TILE = 256
NT = BB // TILE
TILEC = 1024
NTC = BB // TILEC
SPLIT = 4
SUBD = DD // SPLIT
NROWS = BB * SPLIT


def _classify_body(x_ref, wc_ref, bc_ref, lbd_ref, a_ref, rank_ref, cnt_ref):
    x_t = x_ref[...]
    logits = jnp.dot(x_t, wc_ref[...], preferred_element_type=jnp.float32)
    logits = logits + bc_ref[...]
    m = jnp.max(logits, axis=1, keepdims=True)
    lane = jax.lax.broadcasted_iota(jnp.int32, (TILEC, KK), 1)
    amax = jnp.min(jnp.where(logits == m, lane, KK), axis=1, keepdims=True)
    onehot = (lane == amax).astype(jnp.float32)
    ranks = jax.lax.dot(lbd_ref[...], onehot)
    csum = jnp.sum(onehot.reshape(TILEC // TILE, TILE, KK), axis=1)
    cr = jax.lax.broadcasted_iota(jnp.int32, (TILEC // TILE,) * 2, 0)
    cc = jax.lax.broadcasted_iota(jnp.int32, (TILEC // TILE,) * 2, 1)
    lt4 = (cc < cr).astype(jnp.float32)
    cbase = jax.lax.dot(lt4, csum)
    base_tok = jnp.broadcast_to(cbase[:, None, :],
                                (TILEC // TILE, TILE, KK)).reshape(TILEC, KK)
    rank_tok = jnp.sum((ranks + base_tok) * onehot, axis=1, keepdims=True)
    a_ref[...] = amax
    rank_ref[...] = rank_tok
    cnt_ref[...] = jnp.sum(onehot, axis=0, keepdims=True)[None]


def _classify(x, wc, bc, lbd):
    return pl.pallas_call(
        _classify_body,
        grid=(NTC,),
        in_specs=[
            pl.BlockSpec((TILEC, DD), lambda t: (t, 0)),
            pl.BlockSpec((DD, KK), lambda t: (0, 0)),
            pl.BlockSpec((1, KK), lambda t: (0, 0)),
            pl.BlockSpec((TILEC, TILEC), lambda t: (0, 0)),
        ],
        out_specs=[
            pl.BlockSpec((TILEC, 1), lambda t: (t, 0)),
            pl.BlockSpec((TILEC, 1), lambda t: (t, 0)),
            pl.BlockSpec((1, 1, KK), lambda t: (t, 0, 0)),
        ],
        out_shape=[
            jax.ShapeDtypeStruct((BB, 1), jnp.int32),
            jax.ShapeDtypeStruct((BB, 1), jnp.float32),
            jax.ShapeDtypeStruct((NTC, 1, KK), jnp.float32),
        ],
    )(x, wc, bc, lbd)


def _destiny_body(a_ref, rank_ref, cnt_ref, dest4_ref, offs_ref):
    cnts = cnt_ref[:, 0, :]
    ri = jax.lax.broadcasted_iota(jnp.int32, (NTC, NTC), 0)
    ci = jax.lax.broadcasted_iota(jnp.int32, (NTC, NTC), 1)
    ltri = (ci < ri).astype(jnp.float32)
    carry = jax.lax.dot(ltri, cnts,
                        precision=jax.lax.Precision.HIGHEST)
    tot = jnp.sum(cnts, axis=0, keepdims=True)
    er = jax.lax.broadcasted_iota(jnp.int32, (KK, KK), 0)
    ec = jax.lax.broadcasted_iota(jnp.int32, (KK, KK), 1)
    xtri = (er < ec).astype(jnp.float32)
    offs = jax.lax.dot(tot, xtri,
                       precision=jax.lax.Precision.HIGHEST)
    v = offs[None] + carry[:, None, :]
    vtok = jnp.broadcast_to(v, (NTC, TILEC, KK)).reshape(BB, KK)
    a_t = a_ref[...]
    lane = jax.lax.broadcasted_iota(jnp.int32, (BB, KK), 1)
    onehot = (lane == a_t).astype(jnp.float32)
    base = jnp.sum(onehot * vtok, axis=1, keepdims=True)
    dest = (base + rank_ref[...]).astype(jnp.int32)
    sub = jax.lax.broadcasted_iota(jnp.int32, (BB, SPLIT), 1)
    dest4_ref[...] = dest + sub * BB
    lane16 = jax.lax.broadcasted_iota(jnp.int32, (1, 16), 1)
    offs16 = jnp.where(lane16 < KK,
                       jnp.pad(offs, ((0, 0), (0, 8))), BB).astype(jnp.int32)
    offs_ref[...] = offs16


def _destiny(a, rank, cnt):
    return pl.pallas_call(
        _destiny_body,
        in_specs=[
            pl.BlockSpec((BB, 1), lambda: (0, 0)),
            pl.BlockSpec((BB, 1), lambda: (0, 0)),
            pl.BlockSpec((NTC, 1, KK), lambda: (0, 0, 0)),
        ],
        out_specs=[
            pl.BlockSpec((BB, SPLIT), lambda: (0, 0)),
            pl.BlockSpec((1, 16), lambda: (0, 0)),
        ],
        out_shape=[
            jax.ShapeDtypeStruct((BB, SPLIT), jnp.int32),
            jax.ShapeDtypeStruct((1, 16), jnp.int32),
        ],
    )(a, rank, cnt)


_SC_WIN = 128
_NW = BB // _SC_WIN
SUBI = DD // 2 // SPLIT


def _sc_scatter(x, dest_row):
    mesh = plsc.VectorSubcoreMesh(core_axis_name="core",
                                  subcore_axis_name="subcore")

    @functools.partial(
        pl.kernel,
        out_type=jax.ShapeDtypeStruct((NROWS, SUBD), jnp.float32),
        mesh=mesh)
    def run(x_hbm, i_hbm, o_hbm):
        def body(x_vmem, i_vmem):
            pltpu.sync_copy(x_vmem, o_hbm.at[i_vmem.at[0]])

        pltpu.emit_pipeline(
            body,
            grid=(SPLIT, _NW),
            in_specs=[
                pl.BlockSpec((_SC_WIN, SUBD), lambda c, w: (w, c)),
                pl.BlockSpec((1, _SC_WIN), lambda c, w: (0, c * _NW + w)),
            ],
            out_specs=[],
            core_axis_name=("core", "subcore"),
            dimension_semantics=(pltpu.PARALLEL, pltpu.PARALLEL),
        )(x_hbm, i_hbm)

    return run(x, dest_row)


def _sc_gather(r0, r1, r2, r3, dest_plain):
    mesh = plsc.VectorSubcoreMesh(core_axis_name="core",
                                  subcore_axis_name="subcore")

    @functools.partial(
        pl.kernel,
        out_type=jax.ShapeDtypeStruct((BB, DD), jnp.float32),
        mesh=mesh)
    def run(r0_hbm, r1_hbm, r2_hbm, r3_hbm, i_hbm, o_hbm):
        for cplane, r_hbm in enumerate((r0_hbm, r1_hbm, r2_hbm, r3_hbm)):
            def body(i_vmem, o_vmem, r_hbm=r_hbm):
                pltpu.sync_copy(r_hbm.at[i_vmem.at[0]], o_vmem)

            pltpu.emit_pipeline(
                body,
                grid=(_NW,),
                in_specs=[pl.BlockSpec((1, _SC_WIN), lambda w: (0, w))],
                out_specs=[pl.BlockSpec((_SC_WIN, SUBD),
                                        lambda w, cplane=cplane: (w, cplane))],
                core_axis_name=("core", "subcore"),
                dimension_semantics=(pltpu.PARALLEL,),
            )(i_hbm, o_hbm)

    return run(r0, r1, r2, r3, dest_plain)


def _ae_body(offs_ref, x0_ref, x1_ref, x2_ref, x3_ref,
             w1_ref, b1_ref, w2_ref, b2_ref,
             r0_ref, r1_ref, r2_ref, r3_ref):
    t = pl.program_id(0)
    row0 = t * TILE
    rows = jax.lax.broadcasted_iota(jnp.int32, (TILE, 1), 0)
    x_cat = jnp.concatenate(
        [x0_ref[...], x1_ref[...], x2_ref[...], x3_ref[...]],
        axis=1)
    r_refs = (r0_ref, r1_ref, r2_ref, r3_ref)
    for e in range(KK):
        s = jnp.clip(offs_ref[e] - row0, 0, TILE)
        en = jnp.clip(offs_ref[e + 1] - row0, 0, TILE)

        @pl.when(en > s)
        def _():
            h = jnp.dot(x_cat, w1_ref[e], preferred_element_type=jnp.float32)
            h = jax.nn.relu(h + b1_ref[e][None, :])
            r = jnp.dot(h, w2_ref[e], preferred_element_type=jnp.float32)
            r = r + b2_ref[e][None, :]

            @pl.when((s == 0) & (en == TILE))
            def _full():
                for cp in range(SPLIT):
                    r_refs[cp][...] = r[:, cp * SUBD:(cp + 1) * SUBD]

            @pl.when((s > 0) | (en < TILE))
            def _partial():
                mask = (rows >= s) & (rows < en)
                for cp in range(SPLIT):
                    r_refs[cp][...] = jnp.where(
                        mask, r[:, cp * SUBD:(cp + 1) * SUBD], r_refs[cp][...])


def _grouped_ae(offs9, xs4, w1, b1, w2, b2):
    grid_spec = pltpu.PrefetchScalarGridSpec(
        num_scalar_prefetch=1,
        grid=(NT,),
        in_specs=[
            pl.BlockSpec((TILE, SUBD), lambda t, offs: (0 * NT + t, 0)),
            pl.BlockSpec((TILE, SUBD), lambda t, offs: (1 * NT + t, 0)),
            pl.BlockSpec((TILE, SUBD), lambda t, offs: (2 * NT + t, 0)),
            pl.BlockSpec((TILE, SUBD), lambda t, offs: (3 * NT + t, 0)),
            pl.BlockSpec((KK, DD, HH), lambda t, offs: (0, 0, 0)),
            pl.BlockSpec((KK, HH), lambda t, offs: (0, 0)),
            pl.BlockSpec((KK, HH, DD), lambda t, offs: (0, 0, 0)),
            pl.BlockSpec((KK, DD), lambda t, offs: (0, 0)),
        ],
        out_specs=[
            pl.BlockSpec((TILE, SUBD), lambda t, offs: (t, 0)),
            pl.BlockSpec((TILE, SUBD), lambda t, offs: (t, 0)),
            pl.BlockSpec((TILE, SUBD), lambda t, offs: (t, 0)),
            pl.BlockSpec((TILE, SUBD), lambda t, offs: (t, 0)),
        ],
    )
    return pl.pallas_call(
        _ae_body,
        grid_spec=grid_spec,
        out_shape=[jax.ShapeDtypeStruct((BB, SUBD), jnp.float32)] * SPLIT,
    )(offs9, xs4, xs4, xs4, xs4, w1, b1, w2, b2)


def kernel(x, W1, b1, W2, b2, Wc, bc):
    ri = jax.lax.broadcasted_iota(jnp.int32, (TILEC, TILEC), 0)
    ci = jax.lax.broadcasted_iota(jnp.int32, (TILEC, TILEC), 1)
    lbd = ((ci < ri) & (ci // TILE == ri // TILE)).astype(jnp.float32)
    a, rank, cnt = _classify(x, Wc, bc.reshape(1, KK), lbd)
    dest4, offs16 = _destiny(a, rank, cnt)
    dest_row = dest4.T.reshape(1, NROWS)
    dest_plain = dest4[:, 0].reshape(1, BB)
    xs4 = _sc_scatter(x, dest_row)
    r0, r1, r2, r3 = _grouped_ae(offs16.reshape(16), xs4, W1, b1, W2, b2)
    x_out = _sc_gather(r0, r1, r2, r3, dest_plain)
    return (x_out, a.reshape(BB))


# speedup vs baseline: 1.2388x; 1.0783x over previous
import functools

import jax
import jax.numpy as jnp
from jax.experimental import pallas as pl
from jax.experimental.pallas import tpu as pltpu
from jax.experimental.pallas import tpu_sc as plsc

BB, DD, HH, KK = 8192, 1024, 256, 8
TILE = 256
NT = BB // TILE
TILEC = 1024
NTC = BB // TILEC
SPLIT = 4
SUBD = DD // SPLIT
NROWS = BB * SPLIT


def _classify_body(x_ref, wc_ref, bc_ref, a_ref, rank_ref, cnt_ref):
    x_t = x_ref[...]
    logits = jnp.dot(x_t, wc_ref[...], preferred_element_type=jnp.float32)
    logits = logits + bc_ref[...]
    m = jnp.max(logits, axis=1, keepdims=True)
    lane = jax.lax.broadcasted_iota(jnp.int32, (TILEC, KK), 1)
    amax = jnp.min(jnp.where(logits == m, lane, KK), axis=1, keepdims=True)
    onehot = (lane == amax).astype(jnp.float32)
    ri = jax.lax.broadcasted_iota(jnp.int32, (TILEC, TILEC), 0)
    ci = jax.lax.broadcasted_iota(jnp.int32, (TILEC, TILEC), 1)
    lbd = ((ci < ri) & (ci // TILE == ri // TILE)).astype(jnp.float32)
    ranks = jax.lax.dot(lbd, onehot)
    csum = jnp.sum(onehot.reshape(TILEC // TILE, TILE, KK), axis=1)
    cr = jax.lax.broadcasted_iota(jnp.int32, (TILEC // TILE,) * 2, 0)
    cc = jax.lax.broadcasted_iota(jnp.int32, (TILEC // TILE,) * 2, 1)
    lt4 = (cc < cr).astype(jnp.float32)
    cbase = jax.lax.dot(lt4, csum)
    base_tok = jnp.broadcast_to(cbase[:, None, :],
                                (TILEC // TILE, TILE, KK)).reshape(TILEC, KK)
    rank_tok = jnp.sum((ranks + base_tok) * onehot, axis=1, keepdims=True)
    a_ref[...] = amax
    rank_ref[...] = rank_tok
    cnt_ref[...] = jnp.sum(onehot, axis=0, keepdims=True)[None]


def _classify(x, wc, bc):
    return pl.pallas_call(
        _classify_body,
        grid=(NTC,),
        in_specs=[
            pl.BlockSpec((TILEC, DD), lambda t: (t, 0)),
            pl.BlockSpec((DD, KK), lambda t: (0, 0)),
            pl.BlockSpec((1, KK), lambda t: (0, 0)),
        ],
        out_specs=[
            pl.BlockSpec((TILEC, 1), lambda t: (t, 0)),
            pl.BlockSpec((TILEC, 1), lambda t: (t, 0)),
            pl.BlockSpec((1, 1, KK), lambda t: (t, 0, 0)),
        ],
        out_shape=[
            jax.ShapeDtypeStruct((BB, 1), jnp.int32),
            jax.ShapeDtypeStruct((BB, 1), jnp.float32),
            jax.ShapeDtypeStruct((NTC, 1, KK), jnp.float32),
        ],
    )(x, wc, bc)


def _destiny_body(a_ref, rank_ref, cnt_ref, dest4_ref, offs_ref):
    cnts = cnt_ref[:, 0, :]
    ri = jax.lax.broadcasted_iota(jnp.int32, (NTC, NTC), 0)
    ci = jax.lax.broadcasted_iota(jnp.int32, (NTC, NTC), 1)
    ltri = (ci < ri).astype(jnp.float32)
    carry = jax.lax.dot(ltri, cnts,
                        precision=jax.lax.Precision.HIGHEST)
    tot = jnp.sum(cnts, axis=0, keepdims=True)
    er = jax.lax.broadcasted_iota(jnp.int32, (KK, KK), 0)
    ec = jax.lax.broadcasted_iota(jnp.int32, (KK, KK), 1)
    xtri = (er < ec).astype(jnp.float32)
    offs = jax.lax.dot(tot, xtri,
                       precision=jax.lax.Precision.HIGHEST)
    v = offs[None] + carry[:, None, :]
    vtok = jnp.broadcast_to(v, (NTC, TILEC, KK)).reshape(BB, KK)
    a_t = a_ref[...]
    lane = jax.lax.broadcasted_iota(jnp.int32, (BB, KK), 1)
    onehot = (lane == a_t).astype(jnp.float32)
    base = jnp.sum(onehot * vtok, axis=1, keepdims=True)
    dest = (base + rank_ref[...]).astype(jnp.int32)
    sub = jax.lax.broadcasted_iota(jnp.int32, (BB, SPLIT), 1)
    dest4_ref[...] = dest + sub * BB
    offs_ref[...] = offs.astype(jnp.int32)


def _destiny(a, rank, cnt):
    return pl.pallas_call(
        _destiny_body,
        in_specs=[
            pl.BlockSpec((BB, 1), lambda: (0, 0)),
            pl.BlockSpec((BB, 1), lambda: (0, 0)),
            pl.BlockSpec((NTC, 1, KK), lambda: (0, 0, 0)),
        ],
        out_specs=[
            pl.BlockSpec((BB, SPLIT), lambda: (0, 0)),
            pl.BlockSpec((1, KK), lambda: (0, 0)),
        ],
        out_shape=[
            jax.ShapeDtypeStruct((BB, SPLIT), jnp.int32),
            jax.ShapeDtypeStruct((1, KK), jnp.int32),
        ],
    )(a, rank, cnt)


_SC_WIN = 128
_NW = BB // _SC_WIN
SUBI = DD // 2 // SPLIT


def _sc_scatter(x, dest_row):
    mesh = plsc.VectorSubcoreMesh(core_axis_name="core",
                                  subcore_axis_name="subcore")

    @functools.partial(
        pl.kernel,
        out_type=jax.ShapeDtypeStruct((NROWS, SUBD), jnp.float32),
        mesh=mesh)
    def run(x_hbm, i_hbm, o_hbm):
        def body(x_vmem, i_vmem):
            pltpu.sync_copy(x_vmem, o_hbm.at[i_vmem.at[0]])

        pltpu.emit_pipeline(
            body,
            grid=(SPLIT, _NW),
            in_specs=[
                pl.BlockSpec((_SC_WIN, SUBD), lambda c, w: (w, c)),
                pl.BlockSpec((1, _SC_WIN), lambda c, w: (0, c * _NW + w)),
            ],
            out_specs=[],
            core_axis_name=("core", "subcore"),
            dimension_semantics=(pltpu.PARALLEL, pltpu.PARALLEL),
        )(x_hbm, i_hbm)

    return run(x, dest_row)


def _sc_gather(r0, r1, r2, r3, dest_plain):
    mesh = plsc.VectorSubcoreMesh(core_axis_name="core",
                                  subcore_axis_name="subcore")

    @functools.partial(
        pl.kernel,
        out_type=jax.ShapeDtypeStruct((BB, DD), jnp.float32),
        mesh=mesh)
    def run(r0_hbm, r1_hbm, r2_hbm, r3_hbm, i_hbm, o_hbm):
        for cplane, r_hbm in enumerate((r0_hbm, r1_hbm, r2_hbm, r3_hbm)):
            def body(i_vmem, o_vmem, r_hbm=r_hbm):
                pltpu.sync_copy(r_hbm.at[i_vmem.at[0]], o_vmem)

            pltpu.emit_pipeline(
                body,
                grid=(_NW,),
                in_specs=[pl.BlockSpec((1, _SC_WIN), lambda w: (0, w))],
                out_specs=[pl.BlockSpec((_SC_WIN, SUBD),
                                        lambda w, cplane=cplane: (w, cplane))],
                core_axis_name=("core", "subcore"),
                dimension_semantics=(pltpu.PARALLEL,),
            )(i_hbm, o_hbm)

    return run(r0, r1, r2, r3, dest_plain)


ATILE = 512
NTA = BB // ATILE
def _ae_body(offs_ref, x0_ref, x1_ref, x2_ref, x3_ref,
             w1_ref, b1_ref, w2_ref, b2_ref,
             r0_ref, r1_ref, r2_ref, r3_ref):
    t = pl.program_id(0)
    row0 = t * ATILE
    rows = jax.lax.broadcasted_iota(jnp.int32, (ATILE, 1), 0)
    x_cat = jnp.concatenate(
        [x0_ref[...], x1_ref[...], x2_ref[...], x3_ref[...]],
        axis=1)
    r_refs = (r0_ref, r1_ref, r2_ref, r3_ref)
    for e in range(KK):
        s = jnp.clip(offs_ref[e] - row0, 0, ATILE)
        en = jnp.clip(offs_ref[e + 1] - row0, 0, ATILE)

        @pl.when(en > s)
        def _():
            h = jnp.dot(x_cat, w1_ref[e], preferred_element_type=jnp.float32)
            h = jax.nn.relu(h + b1_ref[e][None, :])
            r = jnp.dot(h, w2_ref[e], preferred_element_type=jnp.float32)
            r = r + b2_ref[e][None, :]
            mask = (rows >= s) & (rows < en)
            for cp in range(SPLIT):
                r_refs[cp][...] = jnp.where(
                    mask, r[:, cp * SUBD:(cp + 1) * SUBD], r_refs[cp][...])


def _grouped_ae(offs9, xs4, w1, b1, w2, b2):
    grid_spec = pltpu.PrefetchScalarGridSpec(
        num_scalar_prefetch=1,
        grid=(NTA,),
        in_specs=[
            pl.BlockSpec((ATILE, SUBD), lambda t, offs: (0 * NTA + t, 0)),
            pl.BlockSpec((ATILE, SUBD), lambda t, offs: (1 * NTA + t, 0)),
            pl.BlockSpec((ATILE, SUBD), lambda t, offs: (2 * NTA + t, 0)),
            pl.BlockSpec((ATILE, SUBD), lambda t, offs: (3 * NTA + t, 0)),
            pl.BlockSpec((KK, DD, HH), lambda t, offs: (0, 0, 0)),
            pl.BlockSpec((KK, HH), lambda t, offs: (0, 0)),
            pl.BlockSpec((KK, HH, DD), lambda t, offs: (0, 0, 0)),
            pl.BlockSpec((KK, DD), lambda t, offs: (0, 0)),
        ],
        out_specs=[
            pl.BlockSpec((ATILE, SUBD), lambda t, offs: (t, 0)),
            pl.BlockSpec((ATILE, SUBD), lambda t, offs: (t, 0)),
            pl.BlockSpec((ATILE, SUBD), lambda t, offs: (t, 0)),
            pl.BlockSpec((ATILE, SUBD), lambda t, offs: (t, 0)),
        ],
    )
    return pl.pallas_call(
        _ae_body,
        grid_spec=grid_spec,
        out_shape=[jax.ShapeDtypeStruct((BB, SUBD), jnp.float32)] * SPLIT,
    )(offs9, xs4, xs4, xs4, xs4, w1, b1, w2, b2)


def kernel(x, W1, b1, W2, b2, Wc, bc):
    a, rank, cnt = _classify(x, Wc, bc.reshape(1, KK))
    dest4, offs = _destiny(a, rank, cnt)
    offs9 = jnp.concatenate(
        [offs.reshape(KK), jnp.array([BB], jnp.int32)])
    dest_row = dest4.T.reshape(1, NROWS)
    dest_plain = dest4[:, 0].reshape(1, BB)
    xs4 = _sc_scatter(x, dest_row)
    r0, r1, r2, r3 = _grouped_ae(offs9, xs4, W1, b1, W2, b2)
    x_out = _sc_gather(r0, r1, r2, r3, dest_plain)
    return (x_out, a.reshape(BB))
